# Initial kernel scaffold; baseline (speedup 1.0000x reference)
import jax
import jax.numpy as jnp
from jax.experimental import pallas as pl


def kernel(y_pred, y_true):
    raise NotImplementedError("write your pallas kernel here")


# speedup vs baseline: 1.1384x; 1.1384x over previous
import jax
import jax.numpy as jnp
from jax.experimental import pallas as pl
from jax.experimental.pallas import tpu as pltpu

_B, _A, _C = 64, 8732, 25
_NEG_POS_RATIO = 3.0
_NEG_INF = float("-inf")


def _ssd_loss_kernel(yp_ref, yt_ref, out_ref, negv_ref, acc_ref):
    b = pl.program_id(0)

    @pl.when(b == 0)
    def _init():
        acc_ref[0] = 0.0
        acc_ref[1] = 0.0
        acc_ref[2] = 0.0
        acc_ref[3] = 0.0

    yp = yp_ref[0]
    yt = yt_ref[0]
    ch = jax.lax.broadcasted_iota(jnp.int32, (_A, _C), 1)
    conf_mask = ch < _C - 4
    pos_ch_mask = (ch >= 1) & (ch < _C - 4)
    loc_mask = ch >= _C - 4

    conf_row = -jnp.sum(jnp.where(conf_mask, yt * yp, 0.0), axis=1)
    row_max = jnp.max(jnp.where(pos_ch_mask, yt, _NEG_INF), axis=1)
    pos_row = row_max != 0.0
    neg_row = yt[:, 0] != 0.0

    acc_ref[0] += jnp.sum(pos_row.astype(jnp.float32))
    acc_ref[1] += jnp.sum(neg_row.astype(jnp.float32))
    acc_ref[2] += jnp.sum(jnp.where(pos_row, conf_row, 0.0))

    d = jnp.where(loc_mask & pos_row[:, None], yp - yt, 0.0)
    ad = jnp.abs(d)
    acc_ref[3] += jnp.sum(jnp.where(ad < 1.0, 0.5 * d * d, ad - 0.5))

    negv_ref[b, :] = jnp.where(neg_row, conf_row, _NEG_INF)

    @pl.when(b == _B - 1)
    def _finalize():
        n_pos = acc_ref[0]
        cnt_neg = acc_ref[1]
        k = jnp.minimum(jnp.floor(_NEG_POS_RATIO * n_pos), cnt_neg)

        def _topk_sum():
            vals = negv_ref[...]
            iv = jax.lax.bitcast_convert_type(vals, jnp.int32)
            ikeys = jnp.where(iv >= 0, iv, iv ^ jnp.int32(0x7FFFFFFF))

            cnt_ge0 = jnp.sum((ikeys >= 0).astype(jnp.float32))
            prefix0 = jnp.where(cnt_ge0 >= k, jnp.int32(0),
                                jnp.int32(-2147483648))

            def body(i, prefix):
                bit = jax.lax.shift_left(jnp.int32(1), jnp.int32(30) - i)
                cand = prefix | bit
                cnt = jnp.sum((ikeys >= cand).astype(jnp.float32))
                return jnp.where(cnt >= k, cand, prefix)

            vkey = jax.lax.fori_loop(0, 31, body, prefix0)
            v = jnp.max(jnp.where(ikeys == vkey, vals, _NEG_INF))
            gt = ikeys > vkey
            cnt_gt = jnp.sum(jnp.where(gt, 1.0, 0.0))
            sum_gt = jnp.sum(jnp.where(gt, vals, 0.0))
            return sum_gt + (k - cnt_gt) * v

        topk = jax.lax.cond(k >= 1.0, _topk_sum, lambda: jnp.float32(0.0))
        total = acc_ref[2] + topk + acc_ref[3]
        out_ref[...] = jnp.full((1, 1), total / jnp.maximum(n_pos, 1.0),
                                jnp.float32)


def kernel(y_pred, y_true):
    out = pl.pallas_call(
        _ssd_loss_kernel,
        grid=(_B,),
        in_specs=[
            pl.BlockSpec((1, _A, _C), lambda b: (b, 0, 0)),
            pl.BlockSpec((1, _A, _C), lambda b: (b, 0, 0)),
        ],
        out_specs=pl.BlockSpec((1, 1), lambda b: (0, 0)),
        out_shape=jax.ShapeDtypeStruct((1, 1), jnp.float32),
        scratch_shapes=[
            pltpu.VMEM((_B, _A), jnp.float32),
            pltpu.SMEM((4,), jnp.float32),
        ],
        compiler_params=pltpu.CompilerParams(
            dimension_semantics=("arbitrary",),
        ),
    )(y_pred, y_true)
    return out[0, 0]
